# baseline (speedup 1.0000x reference)
import functools

import jax
import jax.numpy as jnp
from jax import lax
from jax.experimental import pallas as pl
from jax.experimental.pallas import tpu as pltpu
from jax.experimental.pallas import tpu_sc as plsc

VOCAB = 100000
EMB_DIM = 768
BATCH = 16384
NCHUNK = 4
CB = BATCH // NCHUNK


def _sc_gather(table, idx, chunk):
    info = plsc.get_sparse_core_info()
    nw = info.num_cores * info.num_subcores
    b_per_w = CB // nw
    CH = min(b_per_w, 64)
    n_st = b_per_w // CH

    mesh = plsc.VectorSubcoreMesh(core_axis_name="c", subcore_axis_name="s")

    @functools.partial(
        pl.kernel,
        mesh=mesh,
        out_type=jax.ShapeDtypeStruct((CB, EMB_DIM), jnp.float32),
        scratch_types=[
            pltpu.VMEM((b_per_w,), jnp.int32),
            pltpu.VMEM((CH, EMB_DIM), jnp.float32),
            pltpu.VMEM((CH, EMB_DIM), jnp.float32),
            pltpu.SemaphoreType.DMA,
            pltpu.SemaphoreType.DMA,
            pltpu.SemaphoreType.DMA,
        ],
    )
    def gather_kernel(table_hbm, idx_hbm, out_hbm, idx_v, rows_a, rows_b,
                      sem_g, sem_wa, sem_wb):
        wid = lax.axis_index("s") * info.num_cores + lax.axis_index("c")
        base = wid * b_per_w
        pltpu.sync_copy(idx_hbm.at[pl.ds(chunk * CB + base, b_per_w)], idx_v)

        bufs = (rows_a, rows_b)
        wsems = (sem_wa, sem_wb)
        pltpu.async_copy(
            table_hbm.at[idx_v.at[pl.ds(0, CH)]], bufs[0], sem_g
        ).wait()
        for s in range(n_st):
            buf, wsem = bufs[s % 2], wsems[s % 2]
            if s + 1 < n_st:
                nbuf = bufs[(s + 1) % 2]
                if s + 1 >= 2:
                    pltpu.make_async_copy(
                        nbuf, out_hbm.at[pl.ds(0, CH)], wsems[(s + 1) % 2]
                    ).wait()
                gcp = pltpu.async_copy(
                    table_hbm.at[idx_v.at[pl.ds((s + 1) * CH, CH)]], nbuf, sem_g
                )
            pltpu.async_copy(buf, out_hbm.at[pl.ds(base + s * CH, CH)], wsem)
            if s + 1 < n_st:
                gcp.wait()
        for s in range(max(0, n_st - 2), n_st):
            pltpu.make_async_copy(
                bufs[s % 2], out_hbm.at[pl.ds(0, CH)], wsems[s % 2]
            ).wait()

    return gather_kernel(table, idx)


_BM = 512


def _mm_common(x_ref, w_ref, tt_ref, wl_ref, b_ref, o_ref):
    w1 = w_ref[:, :EMB_DIM]
    acc = lax.dot_general(
        x_ref[...].astype(jnp.bfloat16), w1.astype(jnp.bfloat16),
        (((1,), (1,)), ((), ())),
        preferred_element_type=jnp.float32,
    )
    o_ref[...] = acc + tt_ref[...] * wl_ref[...] + b_ref[...]


def _mm_body_first(x_ref, w_ref, tt_ref, wl_ref, b_ref, o_ref):
    _mm_common(x_ref, w_ref, tt_ref, wl_ref, b_ref, o_ref)


def _mm_body_alias(o_hbm_ref, x_ref, w_ref, tt_ref, wl_ref, b_ref, o_ref):
    del o_hbm_ref
    _mm_common(x_ref, w_ref, tt_ref, wl_ref, b_ref, o_ref)


def _tc_linear_chunk(out_buf, emb, tt, W2, wlast, b2, chunk):
    nblk = CB // _BM
    blk0 = chunk * nblk
    data_specs = [
        pl.BlockSpec((_BM, EMB_DIM), lambda i: (i, 0)),
        pl.BlockSpec((EMB_DIM, EMB_DIM + 1), lambda i: (0, 0)),
        pl.BlockSpec((_BM, 1), lambda i, b=blk0: (b + i, 0)),
        pl.BlockSpec((1, EMB_DIM), lambda i: (0, 0)),
        pl.BlockSpec((1, EMB_DIM), lambda i: (0, 0)),
    ]
    if out_buf is None:
        body, in_specs, alias, args = (
            _mm_body_first, data_specs, {}, (emb, W2, tt, wlast, b2))
    else:
        body = _mm_body_alias
        in_specs = [pl.BlockSpec(memory_space=pltpu.MemorySpace.HBM)] + data_specs
        alias = {0: 0}
        args = (out_buf, emb, W2, tt, wlast, b2)
    return pl.pallas_call(
        body,
        grid=(nblk,),
        in_specs=in_specs,
        out_specs=pl.BlockSpec((_BM, EMB_DIM), lambda i, b=blk0: (b + i, 0)),
        out_shape=jax.ShapeDtypeStruct((BATCH, EMB_DIM), jnp.float32),
        input_output_aliases=alias,
    )(*args)


def kernel(qubit, total_time, emb_table, W, b):
    idx = qubit.astype(jnp.int32)
    b2 = b.reshape(1, EMB_DIM)
    wlast = W[:, EMB_DIM].reshape(1, EMB_DIM)

    embs = [_sc_gather(emb_table, idx, c) for c in range(NCHUNK)]
    out = None
    for c in range(NCHUNK):
        out = _tc_linear_chunk(out, embs[c], total_time, W, wlast, b2, c)
    return out


# speedup vs baseline: 1.0575x; 1.0575x over previous
import functools

import jax
import jax.numpy as jnp
from jax import lax
from jax.experimental import pallas as pl
from jax.experimental.pallas import tpu as pltpu
from jax.experimental.pallas import tpu_sc as plsc

VOCAB = 100000
EMB_DIM = 768
BATCH = 16384
NCHUNK = 4
CB = BATCH // NCHUNK


def _sc_gather(table, idx, chunk):
    info = plsc.get_sparse_core_info()
    nw = info.num_cores * info.num_subcores
    b_per_w = CB // nw
    CH = min(b_per_w, 64)
    n_st = b_per_w // CH

    mesh = plsc.VectorSubcoreMesh(core_axis_name="c", subcore_axis_name="s")

    @functools.partial(
        pl.kernel,
        mesh=mesh,
        out_type=jax.ShapeDtypeStruct((CB, EMB_DIM), jnp.float32),
        scratch_types=[
            pltpu.VMEM((b_per_w,), jnp.int32),
            pltpu.VMEM((CH, EMB_DIM), jnp.float32),
            pltpu.VMEM((CH, EMB_DIM), jnp.float32),
            pltpu.SemaphoreType.DMA,
            pltpu.SemaphoreType.DMA,
            pltpu.SemaphoreType.DMA,
        ],
    )
    def gather_kernel(table_hbm, idx_hbm, out_hbm, idx_v, rows_a, rows_b,
                      sem_g, sem_wa, sem_wb):
        wid = lax.axis_index("s") * info.num_cores + lax.axis_index("c")
        base = wid * b_per_w
        pltpu.sync_copy(idx_hbm.at[pl.ds(chunk * CB + base, b_per_w)], idx_v)

        bufs = (rows_a, rows_b)
        wsems = (sem_wa, sem_wb)
        pltpu.async_copy(
            table_hbm.at[idx_v.at[pl.ds(0, CH)]], bufs[0], sem_g
        ).wait()
        for s in range(n_st):
            buf, wsem = bufs[s % 2], wsems[s % 2]
            if s + 1 < n_st:
                nbuf = bufs[(s + 1) % 2]
                if s + 1 >= 2:
                    pltpu.make_async_copy(
                        nbuf, out_hbm.at[pl.ds(0, CH)], wsems[(s + 1) % 2]
                    ).wait()
                gcp = pltpu.async_copy(
                    table_hbm.at[idx_v.at[pl.ds((s + 1) * CH, CH)]], nbuf, sem_g
                )
            pltpu.async_copy(buf, out_hbm.at[pl.ds(base + s * CH, CH)], wsem)
            if s + 1 < n_st:
                gcp.wait()
        for s in range(max(0, n_st - 2), n_st):
            pltpu.make_async_copy(
                bufs[s % 2], out_hbm.at[pl.ds(0, CH)], wsems[s % 2]
            ).wait()

    return gather_kernel(table, idx)


_BM = 2048


def _mm_common(x_ref, w_ref, tt_ref, wl_ref, b_ref, o_ref):
    w1 = w_ref[:, :EMB_DIM]
    acc = lax.dot_general(
        x_ref[...].astype(jnp.bfloat16), w1.astype(jnp.bfloat16),
        (((1,), (1,)), ((), ())),
        preferred_element_type=jnp.float32,
    )
    o_ref[...] = acc + tt_ref[...] * wl_ref[...] + b_ref[...]


def _mm_body_first(x_ref, w_ref, tt_ref, wl_ref, b_ref, o_ref):
    _mm_common(x_ref, w_ref, tt_ref, wl_ref, b_ref, o_ref)


def _mm_body_alias(o_hbm_ref, x_ref, w_ref, tt_ref, wl_ref, b_ref, o_ref):
    del o_hbm_ref
    _mm_common(x_ref, w_ref, tt_ref, wl_ref, b_ref, o_ref)


def _tc_linear_chunk(out_buf, emb, tt, W2, wlast, b2, chunk):
    nblk = CB // _BM
    blk0 = chunk * nblk
    data_specs = [
        pl.BlockSpec((_BM, EMB_DIM), lambda i: (i, 0)),
        pl.BlockSpec((EMB_DIM, EMB_DIM + 1), lambda i: (0, 0)),
        pl.BlockSpec((_BM, 1), lambda i, b=blk0: (b + i, 0)),
        pl.BlockSpec((1, EMB_DIM), lambda i: (0, 0)),
        pl.BlockSpec((1, EMB_DIM), lambda i: (0, 0)),
    ]
    if out_buf is None:
        body, in_specs, alias, args = (
            _mm_body_first, data_specs, {}, (emb, W2, tt, wlast, b2))
    else:
        body = _mm_body_alias
        in_specs = [pl.BlockSpec(memory_space=pltpu.MemorySpace.HBM)] + data_specs
        alias = {0: 0}
        args = (out_buf, emb, W2, tt, wlast, b2)
    return pl.pallas_call(
        body,
        grid=(nblk,),
        in_specs=in_specs,
        out_specs=pl.BlockSpec((_BM, EMB_DIM), lambda i, b=blk0: (b + i, 0)),
        out_shape=jax.ShapeDtypeStruct((BATCH, EMB_DIM), jnp.float32),
        input_output_aliases=alias,
    )(*args)


def kernel(qubit, total_time, emb_table, W, b):
    idx = qubit.astype(jnp.int32)
    b2 = b.reshape(1, EMB_DIM)
    wlast = W[:, EMB_DIM].reshape(1, EMB_DIM)

    embs = [_sc_gather(emb_table, idx, c) for c in range(NCHUNK)]
    out = None
    for c in range(NCHUNK):
        out = _tc_linear_chunk(out, embs[c], total_time, W, wlast, b2, c)
    return out
